# Initial kernel scaffold; baseline (speedup 1.0000x reference)
import jax
import jax.numpy as jnp
from jax.experimental import pallas as pl


def kernel(x, edge_index, edge_label_index, hidden, W_gcn, b_gcn, Wz0, Wz1, bz, Wr0, Wr1, br, Wh0, Wh1, bh, W_lin, b_lin):
    raise NotImplementedError("write your pallas kernel here")


# speedup vs baseline: 11.7542x; 11.7542x over previous
import functools

import jax
import jax.numpy as jnp
from jax import lax
from jax.experimental import pallas as pl
from jax.experimental.pallas import tpu as pltpu
from jax.experimental.pallas import tpu_sc as plsc

N = 10000
D = 256
S = 256
C = 128
E = 160000
EL = 160000

NC = 2
NS = 16
NW = NC * NS

NPAD = 10240
EPAD = 163840
ROWS_PER_TILE = NPAD // NS
EB = 128
DEG_BATCHES = EPAD // NW // EB
AGG_BATCHES = EPAD // NS // EB
DEC_CHUNK = 8
EIDX_ROWS = EPAD // EB


def _mesh():
    return plsc.VectorSubcoreMesh(core_axis_name="c", subcore_axis_name="s",
                                  num_cores=NC, num_subcores=NS)


def _zero_vmem(ref, n):
    zeros = jnp.zeros((16,), jnp.float32)

    def body(i, _):
        ref[pl.ds(i * 16, 16)] = zeros
        return 0

    lax.fori_loop(0, n // 16, body, 0)


def _deg_body(cols_hbm, dp0_hbm, dp1_hbm, colv, onesv, zerov, acc_sh):
    c = lax.axis_index("c")
    s = lax.axis_index("s")
    w = s * NC + c

    _zero_vmem(zerov, ROWS_PER_TILE)
    pltpu.sync_copy(zerov.at[pl.ds(0, ROWS_PER_TILE)],
                    acc_sh.at[pl.ds(s * ROWS_PER_TILE, ROWS_PER_TILE)])

    def fill(i, _):
        onesv[pl.ds(i * 16, 16)] = jnp.ones((16,), jnp.float32)
        return 0

    lax.fori_loop(0, EB // 16, fill, 0)
    pltpu.sync_copy(cols_hbm.at[pl.ds(w * DEG_BATCHES, DEG_BATCHES)], colv)
    plsc.subcore_barrier()

    def body(b, _):
        pltpu.sync_copy(onesv, acc_sh.at[colv.at[b]], add=True)
        return 0

    lax.fori_loop(0, DEG_BATCHES, body, 0)
    plsc.subcore_barrier()

    sl = pl.ds(s * ROWS_PER_TILE, ROWS_PER_TILE)

    @pl.when(c == 0)
    def _():
        pltpu.sync_copy(acc_sh.at[sl], dp0_hbm.at[sl])

    @pl.when(c == 1)
    def _():
        pltpu.sync_copy(acc_sh.at[sl], dp1_hbm.at[sl])


@functools.lru_cache(maxsize=None)
def _deg_kernel():
    return pl.kernel(
        _deg_body,
        out_type=[jax.ShapeDtypeStruct((NPAD,), jnp.float32),
                  jax.ShapeDtypeStruct((NPAD,), jnp.float32)],
        mesh=_mesh(),
        scratch_types=[pltpu.VMEM((DEG_BATCHES, EB), jnp.int32),
                       pltpu.VMEM((EB,), jnp.float32),
                       pltpu.VMEM((ROWS_PER_TILE,), jnp.float32),
                       pltpu.VMEM_SHARED((NPAD,), jnp.float32)],
    )


def _agg_body(rows_hbm, cols_hbm, y0_hbm, y1_hbm, agg0_hbm, agg1_hbm,
              rowsv, colsv, gbuf, acc_sh, sem):
    c = lax.axis_index("c")
    s = lax.axis_index("s")

    zeros = jnp.zeros((16,), jnp.float32)

    def zrow(i, _):
        for k in range(C // 16):
            gbuf[i, pl.ds(k * 16, 16)] = zeros
        return 0

    lax.fori_loop(0, EB, zrow, 0)

    def zb(k, _):
        pltpu.sync_copy(gbuf,
                        acc_sh.at[pl.ds(s * ROWS_PER_TILE + k * EB, EB)])
        return 0

    lax.fori_loop(0, ROWS_PER_TILE // EB, zb, 0)

    pltpu.sync_copy(rows_hbm.at[pl.ds(s * AGG_BATCHES, AGG_BATCHES)], rowsv)
    pltpu.sync_copy(cols_hbm.at[pl.ds(s * AGG_BATCHES, AGG_BATCHES)], colsv)
    plsc.subcore_barrier()

    def body(b, _):
        @pl.when(c == 0)
        def _():
            pltpu.async_copy(y0_hbm.at[rowsv.at[b]], gbuf, sem).wait()

        @pl.when(c == 1)
        def _():
            pltpu.async_copy(y1_hbm.at[rowsv.at[b]], gbuf, sem).wait()

        pltpu.sync_copy(gbuf, acc_sh.at[colsv.at[b]], add=True)
        return 0

    lax.fori_loop(0, AGG_BATCHES, body, 0)
    plsc.subcore_barrier()

    sl = pl.ds(s * ROWS_PER_TILE, ROWS_PER_TILE)

    @pl.when(c == 0)
    def _():
        pltpu.sync_copy(acc_sh.at[sl], agg0_hbm.at[sl])

    @pl.when(c == 1)
    def _():
        pltpu.sync_copy(acc_sh.at[sl], agg1_hbm.at[sl])


@functools.lru_cache(maxsize=None)
def _agg_kernel():
    return pl.kernel(
        _agg_body,
        out_type=[jax.ShapeDtypeStruct((NPAD, C), jnp.float32),
                  jax.ShapeDtypeStruct((NPAD, C), jnp.float32)],
        mesh=_mesh(),
        scratch_types=[pltpu.VMEM((AGG_BATCHES, EB), jnp.int32),
                       pltpu.VMEM((AGG_BATCHES, EB), jnp.int32),
                       pltpu.VMEM((EB, C), jnp.float32),
                       pltpu.VMEM_SHARED((NPAD, C), jnp.float32),
                       pltpu.SemaphoreType.DMA],
    )


def _dec_body(z_hbm, src_hbm, dst_hbm, r_hbm,
              srcv, dstv, sbuf, dbuf, pacc, rbuf, z_sh, sem):
    c = lax.axis_index("c")
    s = lax.axis_index("s")
    w = s * NC + c

    sl = pl.ds(s * ROWS_PER_TILE, ROWS_PER_TILE)
    pltpu.sync_copy(z_hbm.at[sl], z_sh.at[sl])
    plsc.subcore_barrier()

    lanes = lax.iota(jnp.int32, 16)

    def chunk(cc, _):
        pltpu.sync_copy(
            src_hbm.at[pl.ds(w * DEG_BATCHES + cc * DEC_CHUNK, DEC_CHUNK)], srcv)
        pltpu.sync_copy(
            dst_hbm.at[pl.ds(w * DEG_BATCHES + cc * DEC_CHUNK, DEC_CHUNK)], dstv)

        def body(b, _):
            pltpu.async_copy(z_sh.at[srcv.at[b]], sbuf, sem).wait()
            pltpu.async_copy(z_sh.at[dstv.at[b]], dbuf, sem).wait()

            def edge(e, _):
                acc = (sbuf[e, pl.ds(0, 16)] * dbuf[e, pl.ds(0, 16)])
                for k in range(1, C // 16):
                    acc = acc + sbuf[e, pl.ds(k * 16, 16)] * dbuf[e, pl.ds(k * 16, 16)]
                pacc[pl.ds(e * 16, 16)] = acc
                return 0

            lax.fori_loop(0, EB, edge, 0)

            def group(g, _):
                racc = jnp.zeros((16,), jnp.float32)
                for l in range(16):
                    racc = racc + plsc.load_gather(pacc, [g * 256 + lanes * 16 + l])
                rbuf[pl.ds(g * 16, 16)] = racc
                return 0

            lax.fori_loop(0, EB // 16, group, 0)
            pltpu.sync_copy(
                rbuf,
                r_hbm.at[pl.ds(w * DEG_BATCHES * EB + (cc * DEC_CHUNK + b) * EB, EB)])
            return 0

        lax.fori_loop(0, DEC_CHUNK, body, 0)
        return 0

    lax.fori_loop(0, DEG_BATCHES // DEC_CHUNK, chunk, 0)


@functools.lru_cache(maxsize=None)
def _dec_kernel():
    return pl.kernel(
        _dec_body,
        out_type=[jax.ShapeDtypeStruct((EPAD,), jnp.float32)],
        mesh=_mesh(),
        scratch_types=[pltpu.VMEM((DEC_CHUNK, EB), jnp.int32),
                       pltpu.VMEM((DEC_CHUNK, EB), jnp.int32),
                       pltpu.VMEM((EB, C), jnp.float32),
                       pltpu.VMEM((EB, C), jnp.float32),
                       pltpu.VMEM((EB * 16,), jnp.float32),
                       pltpu.VMEM((EB,), jnp.float32),
                       pltpu.VMEM_SHARED((NPAD, C), jnp.float32),
                       pltpu.SemaphoreType.DMA],
        compiler_params=pltpu.CompilerParams(needs_layout_passes=False),
    )


RB = 1024
GRID = NPAD // RB
DR = RB // 128


def _tile_to_col(t):
    r0 = lax.broadcasted_iota(jnp.int32, (RB, DR), 0)
    q0 = lax.broadcasted_iota(jnp.int32, (RB, DR), 1)
    a = jnp.where(r0 // 128 == q0, 1.0, 0.0)
    c2d = jnp.dot(a, t, preferred_element_type=jnp.float32)
    r1 = lax.broadcasted_iota(jnp.int32, (RB, 128), 0)
    l1 = lax.broadcasted_iota(jnp.int32, (RB, 128), 1)
    sel = jnp.where(l1 == r1 % 128, c2d, 0.0)
    return jnp.sum(sel, axis=1, keepdims=True)


def _enc_body(x_ref, w_ref, dp0_ref, dp1_ref, y0_ref, y1_ref, dinv_ref):
    deg = dp0_ref[...] + dp1_ref[...] + 1.0
    dinv = lax.rsqrt(deg)
    dinv_ref[...] = dinv
    xw = jnp.dot(x_ref[...], w_ref[...], preferred_element_type=jnp.float32)
    y = xw * _tile_to_col(dinv)
    y0_ref[...] = y[:, :C]
    y1_ref[...] = y[:, C:]


def _enc_call(x_pad, w_gcn, dp0r, dp1r):
    return pl.pallas_call(
        _enc_body,
        grid=(GRID,),
        in_specs=[
            pl.BlockSpec((RB, D), lambda i: (i, 0)),
            pl.BlockSpec((D, D), lambda i: (0, 0)),
            pl.BlockSpec((DR, 128), lambda i: (i, 0)),
            pl.BlockSpec((DR, 128), lambda i: (i, 0)),
        ],
        out_specs=[
            pl.BlockSpec((RB, C), lambda i: (i, 0)),
            pl.BlockSpec((RB, C), lambda i: (i, 0)),
            pl.BlockSpec((DR, 128), lambda i: (i, 0)),
        ],
        out_shape=[
            jax.ShapeDtypeStruct((NPAD, C), jnp.float32),
            jax.ShapeDtypeStruct((NPAD, C), jnp.float32),
            jax.ShapeDtypeStruct((NPAD // 128, 128), jnp.float32),
        ],
    )(x_pad, w_gcn, dp0r, dp1r)


def _gru_body(agg0_ref, agg1_ref, y0_ref, y1_ref, dinv_ref, hid_ref,
              wz0_ref, wz1_ref, wr0_ref, wr1_ref, wh0_ref, wh1_ref,
              wlin_ref, bias_ref, hn_ref, z_ref):
    dinv = _tile_to_col(dinv_ref[...])
    aggc = jnp.concatenate([agg0_ref[...], agg1_ref[...]], axis=1)
    yc = jnp.concatenate([y0_ref[...], y1_ref[...]], axis=1)
    b_gcn = bias_ref[0:1, :]
    h = dinv * (aggc + yc) + b_gcn

    hid = hid_ref[...]
    cat1 = jnp.concatenate([h, hid], axis=1)
    wz = wz0_ref[...] + wz1_ref[...]
    wr = wr0_ref[...] + wr1_ref[...]
    wh = wh0_ref[...] + wh1_ref[...]
    zg = jax.nn.sigmoid(
        jnp.dot(cat1, wz, preferred_element_type=jnp.float32) + bias_ref[1:2, :])
    rg = jax.nn.sigmoid(
        jnp.dot(cat1, wr, preferred_element_type=jnp.float32) + bias_ref[2:3, :])
    cat2 = jnp.concatenate([h, rg * hid], axis=1)
    ht = jnp.tanh(
        jnp.dot(cat2, wh, preferred_element_type=jnp.float32) + bias_ref[3:4, :])
    hn = zg * hid + (1.0 - zg) * ht
    hn_ref[...] = hn

    hr = jnp.maximum(hn, 0.0)
    logits = (jnp.dot(hr, wlin_ref[...], preferred_element_type=jnp.float32)
              + bias_ref[4:5, :C])
    m = jnp.max(logits, axis=1, keepdims=True)
    ez = jnp.exp(logits - m)
    z_ref[...] = ez / jnp.sum(ez, axis=1, keepdims=True)


def _gru_call(agg0, agg1, y0, y1, dinvr, hid_pad,
              wz0, wz1, wr0, wr1, wh0, wh1, wlin, bias):
    full = lambda a, b: pl.BlockSpec((a, b), lambda i: (0, 0))
    return pl.pallas_call(
        _gru_body,
        grid=(GRID,),
        in_specs=[
            pl.BlockSpec((RB, C), lambda i: (i, 0)),
            pl.BlockSpec((RB, C), lambda i: (i, 0)),
            pl.BlockSpec((RB, C), lambda i: (i, 0)),
            pl.BlockSpec((RB, C), lambda i: (i, 0)),
            pl.BlockSpec((DR, 128), lambda i: (i, 0)),
            pl.BlockSpec((RB, S), lambda i: (i, 0)),
            full(D + S, S), full(D + S, S), full(D + S, S),
            full(D + S, S), full(D + S, S), full(D + S, S),
            full(S, C), full(8, 256),
        ],
        out_specs=[
            pl.BlockSpec((RB, S), lambda i: (i, 0)),
            pl.BlockSpec((RB, C), lambda i: (i, 0)),
        ],
        out_shape=[
            jax.ShapeDtypeStruct((NPAD, S), jnp.float32),
            jax.ShapeDtypeStruct((NPAD, C), jnp.float32),
        ],
    )(agg0, agg1, y0, y1, dinvr, hid_pad,
      wz0, wz1, wr0, wr1, wh0, wh1, wlin, bias)


def kernel(x, edge_index, edge_label_index, hidden, W_gcn, b_gcn,
           Wz0, Wz1, bz, Wr0, Wr1, br, Wh0, Wh1, bh, W_lin, b_lin):
    pad_e = jnp.arange(EPAD - E, dtype=jnp.int32)
    cols_p = jnp.concatenate(
        [edge_index[1], N + (pad_e % (NPAD - N))]).reshape(EIDX_ROWS, EB)
    rows_p = jnp.concatenate(
        [edge_index[0], pad_e % N]).reshape(EIDX_ROWS, EB)
    src_p = jnp.concatenate(
        [edge_label_index[0], pad_e % N]).reshape(EIDX_ROWS, EB)
    dst_p = jnp.concatenate(
        [edge_label_index[1], pad_e % N]).reshape(EIDX_ROWS, EB)

    x_pad = jnp.pad(x, ((0, NPAD - N), (0, 0)))
    hid_pad = jnp.pad(hidden, ((0, NPAD - N), (0, 0)))

    bias = jnp.zeros((8, 256), jnp.float32)
    bias = bias.at[0, :].set(b_gcn).at[1, :].set(bz).at[2, :].set(br)
    bias = bias.at[3, :].set(bh).at[4, :C].set(b_lin)

    dp0, dp1 = _deg_kernel()(cols_p)
    dp0r = dp0.reshape(NPAD // 128, 128)
    dp1r = dp1.reshape(NPAD // 128, 128)

    y0, y1, dinvr = _enc_call(x_pad, W_gcn, dp0r, dp1r)
    agg0, agg1 = _agg_kernel()(rows_p, cols_p, y0, y1)
    hn, z = _gru_call(agg0, agg1, y0, y1, dinvr, hid_pad,
                      Wz0, Wz1, Wr0, Wr1, Wh0, Wh1, W_lin, bias)
    (r_pad,) = _dec_kernel()(z, src_p, dst_p)

    return (r_pad[:EL], hn[:N])


# speedup vs baseline: 13.9589x; 1.1876x over previous
import functools

import jax
import jax.numpy as jnp
from jax import lax
from jax.experimental import pallas as pl
from jax.experimental.pallas import tpu as pltpu
from jax.experimental.pallas import tpu_sc as plsc

N = 10000
D = 256
S = 256
C = 128
E = 160000
EL = 160000

NC = 2
NS = 16
NW = NC * NS

NPAD = 10240
EPAD = 163840
ROWS_PER_TILE = NPAD // NS
EB = 128
DEG_BATCHES = EPAD // NW // EB
AGG_BATCHES = EPAD // NS // EB
DEC_CHUNK = 8
EIDX_ROWS = EPAD // EB


def _mesh():
    return plsc.VectorSubcoreMesh(core_axis_name="c", subcore_axis_name="s",
                                  num_cores=NC, num_subcores=NS)


def _zero_vmem(ref, n):
    zeros = jnp.zeros((16,), jnp.float32)

    def body(i, _):
        ref[pl.ds(i * 16, 16)] = zeros
        return 0

    lax.fori_loop(0, n // 16, body, 0)


def _deg_body(cols_hbm, dp0_hbm, dp1_hbm, colv, onesv, zerov, acc_sh):
    c = lax.axis_index("c")
    s = lax.axis_index("s")
    w = s * NC + c

    _zero_vmem(zerov, ROWS_PER_TILE)
    pltpu.sync_copy(zerov.at[pl.ds(0, ROWS_PER_TILE)],
                    acc_sh.at[pl.ds(s * ROWS_PER_TILE, ROWS_PER_TILE)])

    def fill(i, _):
        onesv[pl.ds(i * 16, 16)] = jnp.ones((16,), jnp.float32)
        return 0

    lax.fori_loop(0, EB // 16, fill, 0)
    pltpu.sync_copy(cols_hbm.at[pl.ds(w * DEG_BATCHES, DEG_BATCHES)], colv)
    plsc.subcore_barrier()

    def body(b, _):
        pltpu.sync_copy(onesv, acc_sh.at[colv.at[b]], add=True)
        return 0

    lax.fori_loop(0, DEG_BATCHES, body, 0)
    plsc.subcore_barrier()

    sl = pl.ds(s * ROWS_PER_TILE, ROWS_PER_TILE)

    @pl.when(c == 0)
    def _():
        pltpu.sync_copy(acc_sh.at[sl], dp0_hbm.at[sl])

    @pl.when(c == 1)
    def _():
        pltpu.sync_copy(acc_sh.at[sl], dp1_hbm.at[sl])


@functools.lru_cache(maxsize=None)
def _deg_kernel():
    return pl.kernel(
        _deg_body,
        out_type=[jax.ShapeDtypeStruct((NPAD,), jnp.float32),
                  jax.ShapeDtypeStruct((NPAD,), jnp.float32)],
        mesh=_mesh(),
        scratch_types=[pltpu.VMEM((DEG_BATCHES, EB), jnp.int32),
                       pltpu.VMEM((EB,), jnp.float32),
                       pltpu.VMEM((ROWS_PER_TILE,), jnp.float32),
                       pltpu.VMEM_SHARED((NPAD,), jnp.float32)],
    )


CH = 16


def _agg_body(rows2_hbm, cols_hbm, ycat_hbm, aggcat_hbm,
              rowcA, rowcB, colsv, gbuf0, gbuf1, acc_sh, gs0, gs1, ss0, ss1):
    c = lax.axis_index("c")
    s = lax.axis_index("s")

    zeros = jnp.zeros((16,), jnp.float32)

    def zrow(i, _):
        for k in range(C // 16):
            gbuf0[i, pl.ds(k * 16, 16)] = zeros
        return 0

    lax.fori_loop(0, EB, zrow, 0)

    def zb(k, _):
        pltpu.sync_copy(gbuf0,
                        acc_sh.at[pl.ds(s * ROWS_PER_TILE + k * EB, EB)])
        return 0

    lax.fori_loop(0, ROWS_PER_TILE // EB, zb, 0)

    pltpu.sync_copy(cols_hbm.at[pl.ds(s * AGG_BATCHES, AGG_BATCHES)], colsv)
    rbase = c * EIDX_ROWS + s * AGG_BATCHES
    rowbufs = [rowcA, rowcB]
    pltpu.sync_copy(rows2_hbm.at[pl.ds(rbase, CH)], rowcA)
    plsc.subcore_barrier()

    gb = [gbuf0, gbuf1]
    gsem = [gs0, gs1]
    ssem = [ss0, ss1]

    def gather(b):
        q, r = divmod(b, CH)
        return pltpu.async_copy(ycat_hbm.at[rowbufs[q % 2].at[r]],
                                gb[b % 2], gsem[b % 2])

    def scatter(b):
        return pltpu.async_copy(gb[b % 2], acc_sh.at[colsv.at[b]],
                                ssem[b % 2], add=True)

    g = {0: gather(0), 1: gather(1)}
    sd = {}
    for b in range(AGG_BATCHES):
        nb = b + 2
        if nb < AGG_BATCHES and nb % CH == 0:
            q = nb // CH
            pltpu.sync_copy(rows2_hbm.at[pl.ds(rbase + q * CH, CH)],
                            rowbufs[q % 2])
        g[b].wait()
        sd[b] = scatter(b)
        if nb < AGG_BATCHES:
            sd[b].wait()
            g[nb] = gather(nb)
    sd[AGG_BATCHES - 2].wait()
    sd[AGG_BATCHES - 1].wait()
    plsc.subcore_barrier()

    pltpu.sync_copy(acc_sh.at[pl.ds(s * ROWS_PER_TILE, ROWS_PER_TILE)],
                    aggcat_hbm.at[pl.ds(c * NPAD + s * ROWS_PER_TILE,
                                        ROWS_PER_TILE)])


@functools.lru_cache(maxsize=None)
def _agg_kernel():
    return pl.kernel(
        _agg_body,
        out_type=[jax.ShapeDtypeStruct((2 * NPAD, C), jnp.float32)],
        mesh=_mesh(),
        scratch_types=[pltpu.VMEM((CH, EB), jnp.int32),
                       pltpu.VMEM((CH, EB), jnp.int32),
                       pltpu.VMEM((AGG_BATCHES, EB), jnp.int32),
                       pltpu.VMEM((EB, C), jnp.float32),
                       pltpu.VMEM((EB, C), jnp.float32),
                       pltpu.VMEM_SHARED((NPAD, C), jnp.float32),
                       pltpu.SemaphoreType.DMA,
                       pltpu.SemaphoreType.DMA,
                       pltpu.SemaphoreType.DMA,
                       pltpu.SemaphoreType.DMA],
    )


def _dec_body(z_hbm, src_hbm, dst_hbm, r_hbm,
              srcv, dstv, sbuf, dbuf, pacc, rbuf, z_sh, sem):
    c = lax.axis_index("c")
    s = lax.axis_index("s")
    w = s * NC + c

    sl = pl.ds(s * ROWS_PER_TILE, ROWS_PER_TILE)
    pltpu.sync_copy(z_hbm.at[sl], z_sh.at[sl])
    plsc.subcore_barrier()

    lanes = lax.iota(jnp.int32, 16)

    def chunk(cc, _):
        pltpu.sync_copy(
            src_hbm.at[pl.ds(w * DEG_BATCHES + cc * DEC_CHUNK, DEC_CHUNK)], srcv)
        pltpu.sync_copy(
            dst_hbm.at[pl.ds(w * DEG_BATCHES + cc * DEC_CHUNK, DEC_CHUNK)], dstv)

        def body(b, _):
            pltpu.async_copy(z_sh.at[srcv.at[b]], sbuf, sem).wait()
            pltpu.async_copy(z_sh.at[dstv.at[b]], dbuf, sem).wait()

            def edge(e, _):
                acc = (sbuf[e, pl.ds(0, 16)] * dbuf[e, pl.ds(0, 16)])
                for k in range(1, C // 16):
                    acc = acc + sbuf[e, pl.ds(k * 16, 16)] * dbuf[e, pl.ds(k * 16, 16)]
                pacc[pl.ds(e * 16, 16)] = acc
                return 0

            lax.fori_loop(0, EB, edge, 0)

            def group(g, _):
                racc = jnp.zeros((16,), jnp.float32)
                for l in range(16):
                    racc = racc + plsc.load_gather(pacc, [g * 256 + lanes * 16 + l])
                rbuf[pl.ds(g * 16, 16)] = racc
                return 0

            lax.fori_loop(0, EB // 16, group, 0)
            pltpu.sync_copy(
                rbuf,
                r_hbm.at[pl.ds(w * DEG_BATCHES * EB + (cc * DEC_CHUNK + b) * EB, EB)])
            return 0

        lax.fori_loop(0, DEC_CHUNK, body, 0)
        return 0

    lax.fori_loop(0, DEG_BATCHES // DEC_CHUNK, chunk, 0)


@functools.lru_cache(maxsize=None)
def _dec_kernel():
    return pl.kernel(
        _dec_body,
        out_type=[jax.ShapeDtypeStruct((EPAD,), jnp.float32)],
        mesh=_mesh(),
        scratch_types=[pltpu.VMEM((DEC_CHUNK, EB), jnp.int32),
                       pltpu.VMEM((DEC_CHUNK, EB), jnp.int32),
                       pltpu.VMEM((EB, C), jnp.float32),
                       pltpu.VMEM((EB, C), jnp.float32),
                       pltpu.VMEM((EB * 16,), jnp.float32),
                       pltpu.VMEM((EB,), jnp.float32),
                       pltpu.VMEM_SHARED((NPAD, C), jnp.float32),
                       pltpu.SemaphoreType.DMA],
        compiler_params=pltpu.CompilerParams(needs_layout_passes=False),
    )


RB = 1024
GRID = NPAD // RB
DR = RB // 128


def _tile_to_col(t):
    r0 = lax.broadcasted_iota(jnp.int32, (RB, DR), 0)
    q0 = lax.broadcasted_iota(jnp.int32, (RB, DR), 1)
    a = jnp.where(r0 // 128 == q0, 1.0, 0.0)
    c2d = jnp.dot(a, t, preferred_element_type=jnp.float32)
    r1 = lax.broadcasted_iota(jnp.int32, (RB, 128), 0)
    l1 = lax.broadcasted_iota(jnp.int32, (RB, 128), 1)
    sel = jnp.where(l1 == r1 % 128, c2d, 0.0)
    return jnp.sum(sel, axis=1, keepdims=True)


def _enc_body(x_ref, w_ref, dp0_ref, dp1_ref, y_ref, dinv_ref):
    deg = dp0_ref[...] + dp1_ref[...] + 1.0
    dinv = lax.rsqrt(deg)
    dinv_ref[...] = dinv
    xw = jnp.dot(x_ref[...], w_ref[...], preferred_element_type=jnp.float32)
    y_ref[...] = xw * _tile_to_col(dinv)


def _enc_call(x_pad, w_gcn, dp0r, dp1r):
    return pl.pallas_call(
        _enc_body,
        grid=(GRID, 2),
        in_specs=[
            pl.BlockSpec((RB, D), lambda i, h: (i, 0)),
            pl.BlockSpec((D, C), lambda i, h: (0, h)),
            pl.BlockSpec((DR, 128), lambda i, h: (i, 0)),
            pl.BlockSpec((DR, 128), lambda i, h: (i, 0)),
        ],
        out_specs=[
            pl.BlockSpec((RB, C), lambda i, h: (h * GRID + i, 0)),
            pl.BlockSpec((DR, 128), lambda i, h: (i, 0)),
        ],
        out_shape=[
            jax.ShapeDtypeStruct((2 * NPAD, C), jnp.float32),
            jax.ShapeDtypeStruct((NPAD // 128, 128), jnp.float32),
        ],
    )(x_pad, w_gcn, dp0r, dp1r)


def _gru_body(agg0_ref, agg1_ref, y0_ref, y1_ref, dinv_ref, hid_ref,
              wz0_ref, wz1_ref, wr0_ref, wr1_ref, wh0_ref, wh1_ref,
              wlin_ref, bias_ref, hn_ref, z_ref):
    dinv = _tile_to_col(dinv_ref[...])
    aggc = jnp.concatenate([agg0_ref[...], agg1_ref[...]], axis=1)
    yc = jnp.concatenate([y0_ref[...], y1_ref[...]], axis=1)
    b_gcn = bias_ref[0:1, :]
    h = dinv * (aggc + yc) + b_gcn

    hid = hid_ref[...]
    cat1 = jnp.concatenate([h, hid], axis=1)
    wz = wz0_ref[...] + wz1_ref[...]
    wr = wr0_ref[...] + wr1_ref[...]
    wh = wh0_ref[...] + wh1_ref[...]
    zg = jax.nn.sigmoid(
        jnp.dot(cat1, wz, preferred_element_type=jnp.float32) + bias_ref[1:2, :])
    rg = jax.nn.sigmoid(
        jnp.dot(cat1, wr, preferred_element_type=jnp.float32) + bias_ref[2:3, :])
    cat2 = jnp.concatenate([h, rg * hid], axis=1)
    ht = jnp.tanh(
        jnp.dot(cat2, wh, preferred_element_type=jnp.float32) + bias_ref[3:4, :])
    hn = zg * hid + (1.0 - zg) * ht
    hn_ref[...] = hn

    hr = jnp.maximum(hn, 0.0)
    logits = (jnp.dot(hr, wlin_ref[...], preferred_element_type=jnp.float32)
              + bias_ref[4:5, :C])
    m = jnp.max(logits, axis=1, keepdims=True)
    ez = jnp.exp(logits - m)
    z_ref[...] = ez / jnp.sum(ez, axis=1, keepdims=True)


def _gru_call(aggcat, ycat, dinvr, hid_pad,
              wz0, wz1, wr0, wr1, wh0, wh1, wlin, bias):
    full = lambda a, b: pl.BlockSpec((a, b), lambda i: (0, 0))
    return pl.pallas_call(
        _gru_body,
        grid=(GRID,),
        in_specs=[
            pl.BlockSpec((RB, C), lambda i: (i, 0)),
            pl.BlockSpec((RB, C), lambda i: (GRID + i, 0)),
            pl.BlockSpec((RB, C), lambda i: (i, 0)),
            pl.BlockSpec((RB, C), lambda i: (GRID + i, 0)),
            pl.BlockSpec((DR, 128), lambda i: (i, 0)),
            pl.BlockSpec((RB, S), lambda i: (i, 0)),
            full(D + S, S), full(D + S, S), full(D + S, S),
            full(D + S, S), full(D + S, S), full(D + S, S),
            full(S, C), full(8, 256),
        ],
        out_specs=[
            pl.BlockSpec((RB, S), lambda i: (i, 0)),
            pl.BlockSpec((RB, C), lambda i: (i, 0)),
        ],
        out_shape=[
            jax.ShapeDtypeStruct((NPAD, S), jnp.float32),
            jax.ShapeDtypeStruct((NPAD, C), jnp.float32),
        ],
    )(aggcat, aggcat, ycat, ycat, dinvr, hid_pad,
      wz0, wz1, wr0, wr1, wh0, wh1, wlin, bias)


def kernel(x, edge_index, edge_label_index, hidden, W_gcn, b_gcn,
           Wz0, Wz1, bz, Wr0, Wr1, br, Wh0, Wh1, bh, W_lin, b_lin):
    pad_e = jnp.arange(EPAD - E, dtype=jnp.int32)
    cols_p = jnp.concatenate(
        [edge_index[1], N + (pad_e % (NPAD - N))]).reshape(EIDX_ROWS, EB)
    rows_p = jnp.concatenate(
        [edge_index[0], pad_e % N]).reshape(EIDX_ROWS, EB)
    src_p = jnp.concatenate(
        [edge_label_index[0], pad_e % N]).reshape(EIDX_ROWS, EB)
    dst_p = jnp.concatenate(
        [edge_label_index[1], pad_e % N]).reshape(EIDX_ROWS, EB)

    x_pad = jnp.pad(x, ((0, NPAD - N), (0, 0)))
    hid_pad = jnp.pad(hidden, ((0, NPAD - N), (0, 0)))

    bias = jnp.zeros((8, 256), jnp.float32)
    bias = bias.at[0, :].set(b_gcn).at[1, :].set(bz).at[2, :].set(br)
    bias = bias.at[3, :].set(bh).at[4, :C].set(b_lin)

    rows2 = jnp.concatenate([rows_p, rows_p + NPAD])

    dp0, dp1 = _deg_kernel()(cols_p)
    dp0r = dp0.reshape(NPAD // 128, 128)
    dp1r = dp1.reshape(NPAD // 128, 128)

    ycat, dinvr = _enc_call(x_pad, W_gcn, dp0r, dp1r)
    (aggcat,) = _agg_kernel()(rows2, cols_p, ycat)
    hn, z = _gru_call(aggcat, ycat, dinvr, hid_pad,
                      Wz0, Wz1, Wr0, Wr1, Wh0, Wh1, W_lin, bias)
    (r_pad,) = _dec_kernel()(z, src_p, dst_p)

    return (r_pad[:EL], hn[:N])


# speedup vs baseline: 14.9793x; 1.0731x over previous
import functools

import jax
import jax.numpy as jnp
from jax import lax
from jax.experimental import pallas as pl
from jax.experimental.pallas import tpu as pltpu
from jax.experimental.pallas import tpu_sc as plsc

N = 10000
D = 256
S = 256
C = 128
E = 160000
EL = 160000

NC = 2
NS = 16
NW = NC * NS

NPAD = 10240
EPAD = 163840
ROWS_PER_TILE = NPAD // NS
EB = 128
DEG_BATCHES = EPAD // NW // EB
AGG_BATCHES = EPAD // NS // EB
DEC_CHUNK = 8
EIDX_ROWS = EPAD // EB


def _mesh():
    return plsc.VectorSubcoreMesh(core_axis_name="c", subcore_axis_name="s",
                                  num_cores=NC, num_subcores=NS)


def _zero_vmem(ref, n):
    zeros = jnp.zeros((16,), jnp.float32)

    def body(i, _):
        ref[pl.ds(i * 16, 16)] = zeros
        return 0

    lax.fori_loop(0, n // 16, body, 0)


def _deg_body(cols_hbm, dp0_hbm, dp1_hbm, colv, onesv, zerov, acc_sh):
    c = lax.axis_index("c")
    s = lax.axis_index("s")
    w = s * NC + c

    _zero_vmem(zerov, ROWS_PER_TILE)
    pltpu.sync_copy(zerov.at[pl.ds(0, ROWS_PER_TILE)],
                    acc_sh.at[pl.ds(s * ROWS_PER_TILE, ROWS_PER_TILE)])

    def fill(i, _):
        onesv[pl.ds(i * 16, 16)] = jnp.ones((16,), jnp.float32)
        return 0

    lax.fori_loop(0, EB // 16, fill, 0)
    pltpu.sync_copy(cols_hbm.at[pl.ds(w * DEG_BATCHES, DEG_BATCHES)], colv)
    plsc.subcore_barrier()

    def body(b, _):
        pltpu.sync_copy(onesv, acc_sh.at[colv.at[b]], add=True)
        return 0

    lax.fori_loop(0, DEG_BATCHES, body, 0)
    plsc.subcore_barrier()

    sl = pl.ds(s * ROWS_PER_TILE, ROWS_PER_TILE)

    @pl.when(c == 0)
    def _():
        pltpu.sync_copy(acc_sh.at[sl], dp0_hbm.at[sl])

    @pl.when(c == 1)
    def _():
        pltpu.sync_copy(acc_sh.at[sl], dp1_hbm.at[sl])


@functools.lru_cache(maxsize=None)
def _deg_kernel():
    return pl.kernel(
        _deg_body,
        out_type=[jax.ShapeDtypeStruct((NPAD,), jnp.float32),
                  jax.ShapeDtypeStruct((NPAD,), jnp.float32)],
        mesh=_mesh(),
        scratch_types=[pltpu.VMEM((DEG_BATCHES, EB), jnp.int32),
                       pltpu.VMEM((EB,), jnp.float32),
                       pltpu.VMEM((ROWS_PER_TILE,), jnp.float32),
                       pltpu.VMEM_SHARED((NPAD,), jnp.float32)],
    )


CH = 16


def _agg_body(rows2_hbm, cols_hbm, ycat_hbm, aggcat_hbm,
              rowcA, rowcB, colsv, gbuf0, gbuf1, acc_sh, gs0, gs1, ss0, ss1):
    c = lax.axis_index("c")
    s = lax.axis_index("s")

    zeros = jnp.zeros((16,), jnp.float32)

    def zrow(i, _):
        for k in range(C // 16):
            gbuf0[i, pl.ds(k * 16, 16)] = zeros
        return 0

    lax.fori_loop(0, EB, zrow, 0)

    def zb(k, _):
        pltpu.sync_copy(gbuf0,
                        acc_sh.at[pl.ds(s * ROWS_PER_TILE + k * EB, EB)])
        return 0

    lax.fori_loop(0, ROWS_PER_TILE // EB, zb, 0)

    pltpu.sync_copy(cols_hbm.at[pl.ds(s * AGG_BATCHES, AGG_BATCHES)], colsv)
    rbase = c * EIDX_ROWS + s * AGG_BATCHES
    rowbufs = [rowcA, rowcB]
    pltpu.sync_copy(rows2_hbm.at[pl.ds(rbase, CH)], rowcA)
    plsc.subcore_barrier()

    gb = [gbuf0, gbuf1]
    gsem = [gs0, gs1]
    ssem = [ss0, ss1]

    def gather(b):
        q, r = divmod(b, CH)
        return pltpu.async_copy(ycat_hbm.at[rowbufs[q % 2].at[r]],
                                gb[b % 2], gsem[b % 2])

    def scatter(b):
        return pltpu.async_copy(gb[b % 2], acc_sh.at[colsv.at[b]],
                                ssem[b % 2], add=True)

    g = {0: gather(0), 1: gather(1)}
    sd = {}
    for b in range(AGG_BATCHES):
        nb = b + 2
        if nb < AGG_BATCHES and nb % CH == 0:
            q = nb // CH
            pltpu.sync_copy(rows2_hbm.at[pl.ds(rbase + q * CH, CH)],
                            rowbufs[q % 2])
        g[b].wait()
        sd[b] = scatter(b)
        if nb < AGG_BATCHES:
            sd[b].wait()
            g[nb] = gather(nb)
    sd[AGG_BATCHES - 2].wait()
    sd[AGG_BATCHES - 1].wait()
    plsc.subcore_barrier()

    pltpu.sync_copy(acc_sh.at[pl.ds(s * ROWS_PER_TILE, ROWS_PER_TILE)],
                    aggcat_hbm.at[pl.ds(c * NPAD + s * ROWS_PER_TILE,
                                        ROWS_PER_TILE)])


@functools.lru_cache(maxsize=None)
def _agg_kernel():
    return pl.kernel(
        _agg_body,
        out_type=[jax.ShapeDtypeStruct((2 * NPAD, C), jnp.float32)],
        mesh=_mesh(),
        scratch_types=[pltpu.VMEM((CH, EB), jnp.int32),
                       pltpu.VMEM((CH, EB), jnp.int32),
                       pltpu.VMEM((AGG_BATCHES, EB), jnp.int32),
                       pltpu.VMEM((EB, C), jnp.float32),
                       pltpu.VMEM((EB, C), jnp.float32),
                       pltpu.VMEM_SHARED((NPAD, C), jnp.float32),
                       pltpu.SemaphoreType.DMA,
                       pltpu.SemaphoreType.DMA,
                       pltpu.SemaphoreType.DMA,
                       pltpu.SemaphoreType.DMA],
    )


EB2 = 64
DEC_BATCHES = EPAD // NW // EB2
DEC_ROWS = EPAD // EB2
DEC_CH = 40


def _dec_body(z_hbm, src_hbm, dst_hbm, r_hbm,
              srcv, dstv, sb0, db0, sb1, db1, pacc, rbuf, z_sh,
              sem_s0, sem_d0, sem_s1, sem_d1):
    c = lax.axis_index("c")
    s = lax.axis_index("s")
    w = s * NC + c

    sl = pl.ds(s * ROWS_PER_TILE, ROWS_PER_TILE)
    pltpu.sync_copy(z_hbm.at[sl], z_sh.at[sl])
    plsc.subcore_barrier()

    lanes = lax.iota(jnp.int32, 16)
    obase = w * DEC_BATCHES * EB2

    def compute(b, sbuf, dbuf):
        def edge(e, _):
            acc = (sbuf[e, pl.ds(0, 16)] * dbuf[e, pl.ds(0, 16)])
            for k in range(1, C // 16):
                acc = acc + sbuf[e, pl.ds(k * 16, 16)] * dbuf[e, pl.ds(k * 16, 16)]
            pacc[pl.ds(e * 16, 16)] = acc
            return 0

        lax.fori_loop(0, EB2, edge, 0, unroll=4)

        def group(g, _):
            racc = jnp.zeros((16,), jnp.float32)
            for l in range(16):
                racc = racc + plsc.load_gather(pacc, [g * 256 + lanes * 16 + l])
            rbuf[pl.ds(g * 16, 16)] = racc
            return 0

        lax.fori_loop(0, EB2 // 16, group, 0)
        pltpu.sync_copy(rbuf, r_hbm.at[pl.ds(obase + b * EB2, EB2)])

    def chunk(q, _):
        pltpu.sync_copy(
            src_hbm.at[pl.ds(w * DEC_BATCHES + q * DEC_CH, DEC_CH)], srcv)
        pltpu.sync_copy(
            dst_hbm.at[pl.ds(w * DEC_BATCHES + q * DEC_CH, DEC_CH)], dstv)

        def body(t2, _):
            lb = 2 * t2
            b = q * DEC_CH + lb
            g_s0 = pltpu.async_copy(z_sh.at[srcv.at[lb]], sb0, sem_s0)
            g_d0 = pltpu.async_copy(z_sh.at[dstv.at[lb]], db0, sem_d0)
            g_s1 = pltpu.async_copy(z_sh.at[srcv.at[lb + 1]], sb1, sem_s1)
            g_d1 = pltpu.async_copy(z_sh.at[dstv.at[lb + 1]], db1, sem_d1)
            g_s0.wait()
            g_d0.wait()
            compute(b, sb0, db0)
            g_s1.wait()
            g_d1.wait()
            compute(b + 1, sb1, db1)
            return 0

        lax.fori_loop(0, DEC_CH // 2, body, 0)
        return 0

    lax.fori_loop(0, DEC_BATCHES // DEC_CH, chunk, 0)


@functools.lru_cache(maxsize=None)
def _dec_kernel():
    return pl.kernel(
        _dec_body,
        out_type=[jax.ShapeDtypeStruct((EPAD,), jnp.float32)],
        mesh=_mesh(),
        scratch_types=[pltpu.VMEM((DEC_CH, EB2), jnp.int32),
                       pltpu.VMEM((DEC_CH, EB2), jnp.int32),
                       pltpu.VMEM((EB2, C), jnp.float32),
                       pltpu.VMEM((EB2, C), jnp.float32),
                       pltpu.VMEM((EB2, C), jnp.float32),
                       pltpu.VMEM((EB2, C), jnp.float32),
                       pltpu.VMEM((EB2 * 16,), jnp.float32),
                       pltpu.VMEM((EB2,), jnp.float32),
                       pltpu.VMEM_SHARED((NPAD, C), jnp.float32),
                       pltpu.SemaphoreType.DMA,
                       pltpu.SemaphoreType.DMA,
                       pltpu.SemaphoreType.DMA,
                       pltpu.SemaphoreType.DMA],
        compiler_params=pltpu.CompilerParams(needs_layout_passes=False),
    )


RB = 1024
GRID = NPAD // RB
DR = RB // 128


def _tile_to_col(t):
    r0 = lax.broadcasted_iota(jnp.int32, (RB, DR), 0)
    q0 = lax.broadcasted_iota(jnp.int32, (RB, DR), 1)
    a = jnp.where(r0 // 128 == q0, 1.0, 0.0)
    c2d = jnp.dot(a, t, preferred_element_type=jnp.float32)
    r1 = lax.broadcasted_iota(jnp.int32, (RB, 128), 0)
    l1 = lax.broadcasted_iota(jnp.int32, (RB, 128), 1)
    sel = jnp.where(l1 == r1 % 128, c2d, 0.0)
    return jnp.sum(sel, axis=1, keepdims=True)


def _enc_body(x_ref, w_ref, dp0_ref, dp1_ref, y_ref, dinv_ref):
    deg = dp0_ref[...] + dp1_ref[...] + 1.0
    dinv = lax.rsqrt(deg)
    dinv_ref[...] = dinv
    xw = jnp.dot(x_ref[...], w_ref[...], preferred_element_type=jnp.float32)
    y_ref[...] = xw * _tile_to_col(dinv)


def _enc_call(x_pad, w_gcn, dp0r, dp1r):
    return pl.pallas_call(
        _enc_body,
        grid=(GRID, 2),
        in_specs=[
            pl.BlockSpec((RB, D), lambda i, h: (i, 0)),
            pl.BlockSpec((D, C), lambda i, h: (0, h)),
            pl.BlockSpec((DR, 128), lambda i, h: (i, 0)),
            pl.BlockSpec((DR, 128), lambda i, h: (i, 0)),
        ],
        out_specs=[
            pl.BlockSpec((RB, C), lambda i, h: (h * GRID + i, 0)),
            pl.BlockSpec((DR, 128), lambda i, h: (i, 0)),
        ],
        out_shape=[
            jax.ShapeDtypeStruct((2 * NPAD, C), jnp.float32),
            jax.ShapeDtypeStruct((NPAD // 128, 128), jnp.float32),
        ],
    )(x_pad, w_gcn, dp0r, dp1r)


def _gru_body(agg0_ref, agg1_ref, y0_ref, y1_ref, dinv_ref, hid_ref,
              wz0_ref, wz1_ref, wr0_ref, wr1_ref, wh0_ref, wh1_ref,
              wlin_ref, bias_ref, hn_ref, z_ref):
    dinv = _tile_to_col(dinv_ref[...])
    aggc = jnp.concatenate([agg0_ref[...], agg1_ref[...]], axis=1)
    yc = jnp.concatenate([y0_ref[...], y1_ref[...]], axis=1)
    b_gcn = bias_ref[0:1, :]
    h = dinv * (aggc + yc) + b_gcn

    hid = hid_ref[...]
    cat1 = jnp.concatenate([h, hid], axis=1)
    wz = wz0_ref[...] + wz1_ref[...]
    wr = wr0_ref[...] + wr1_ref[...]
    wh = wh0_ref[...] + wh1_ref[...]
    zg = jax.nn.sigmoid(
        jnp.dot(cat1, wz, preferred_element_type=jnp.float32) + bias_ref[1:2, :])
    rg = jax.nn.sigmoid(
        jnp.dot(cat1, wr, preferred_element_type=jnp.float32) + bias_ref[2:3, :])
    cat2 = jnp.concatenate([h, rg * hid], axis=1)
    ht = jnp.tanh(
        jnp.dot(cat2, wh, preferred_element_type=jnp.float32) + bias_ref[3:4, :])
    hn = zg * hid + (1.0 - zg) * ht
    hn_ref[...] = hn

    hr = jnp.maximum(hn, 0.0)
    logits = (jnp.dot(hr, wlin_ref[...], preferred_element_type=jnp.float32)
              + bias_ref[4:5, :C])
    m = jnp.max(logits, axis=1, keepdims=True)
    ez = jnp.exp(logits - m)
    z_ref[...] = ez / jnp.sum(ez, axis=1, keepdims=True)


def _gru_call(aggcat, ycat, dinvr, hid_pad,
              wz0, wz1, wr0, wr1, wh0, wh1, wlin, bias):
    full = lambda a, b: pl.BlockSpec((a, b), lambda i: (0, 0))
    return pl.pallas_call(
        _gru_body,
        grid=(GRID,),
        in_specs=[
            pl.BlockSpec((RB, C), lambda i: (i, 0)),
            pl.BlockSpec((RB, C), lambda i: (GRID + i, 0)),
            pl.BlockSpec((RB, C), lambda i: (i, 0)),
            pl.BlockSpec((RB, C), lambda i: (GRID + i, 0)),
            pl.BlockSpec((DR, 128), lambda i: (i, 0)),
            pl.BlockSpec((RB, S), lambda i: (i, 0)),
            full(D + S, S), full(D + S, S), full(D + S, S),
            full(D + S, S), full(D + S, S), full(D + S, S),
            full(S, C), full(8, 256),
        ],
        out_specs=[
            pl.BlockSpec((RB, S), lambda i: (i, 0)),
            pl.BlockSpec((RB, C), lambda i: (i, 0)),
        ],
        out_shape=[
            jax.ShapeDtypeStruct((NPAD, S), jnp.float32),
            jax.ShapeDtypeStruct((NPAD, C), jnp.float32),
        ],
    )(aggcat, aggcat, ycat, ycat, dinvr, hid_pad,
      wz0, wz1, wr0, wr1, wh0, wh1, wlin, bias)


def kernel(x, edge_index, edge_label_index, hidden, W_gcn, b_gcn,
           Wz0, Wz1, bz, Wr0, Wr1, br, Wh0, Wh1, bh, W_lin, b_lin):
    pad_e = jnp.arange(EPAD - E, dtype=jnp.int32)
    cols_p = jnp.concatenate(
        [edge_index[1], N + (pad_e % (NPAD - N))]).reshape(EIDX_ROWS, EB)
    rows_p = jnp.concatenate(
        [edge_index[0], pad_e % N]).reshape(EIDX_ROWS, EB)
    src_p = jnp.concatenate(
        [edge_label_index[0], pad_e % N]).reshape(DEC_ROWS, EB2)
    dst_p = jnp.concatenate(
        [edge_label_index[1], pad_e % N]).reshape(DEC_ROWS, EB2)

    x_pad = jnp.pad(x, ((0, NPAD - N), (0, 0)))
    hid_pad = jnp.pad(hidden, ((0, NPAD - N), (0, 0)))

    bias = jnp.zeros((8, 256), jnp.float32)
    bias = bias.at[0, :].set(b_gcn).at[1, :].set(bz).at[2, :].set(br)
    bias = bias.at[3, :].set(bh).at[4, :C].set(b_lin)

    rows2 = jnp.concatenate([rows_p, rows_p + NPAD])

    dp0, dp1 = _deg_kernel()(cols_p)
    dp0r = dp0.reshape(NPAD // 128, 128)
    dp1r = dp1.reshape(NPAD // 128, 128)

    ycat, dinvr = _enc_call(x_pad, W_gcn, dp0r, dp1r)
    (aggcat,) = _agg_kernel()(rows2, cols_p, ycat)
    hn, z = _gru_call(aggcat, ycat, dinvr, hid_pad,
                      Wz0, Wz1, Wr0, Wr1, Wh0, Wh1, W_lin, bias)
    (r_pad,) = _dec_kernel()(z, src_p, dst_p)

    return (r_pad[:EL], hn[:N])


# speedup vs baseline: 15.0387x; 1.0040x over previous
import functools

import jax
import jax.numpy as jnp
from jax import lax
from jax.experimental import pallas as pl
from jax.experimental.pallas import tpu as pltpu
from jax.experimental.pallas import tpu_sc as plsc

N = 10000
D = 256
S = 256
C = 128
E = 160000
EL = 160000

NC = 2
NS = 16
NW = NC * NS

NPAD = 10240
EPAD = 163840
ROWS_PER_TILE = NPAD // NS
EB = 128
DEG_BATCHES = EPAD // NW // EB
AGG_BATCHES = EPAD // NS // EB
DEC_CHUNK = 8
EIDX_ROWS = EPAD // EB


def _mesh():
    return plsc.VectorSubcoreMesh(core_axis_name="c", subcore_axis_name="s",
                                  num_cores=NC, num_subcores=NS)


def _zero_vmem(ref, n):
    zeros = jnp.zeros((16,), jnp.float32)

    def body(i, _):
        ref[pl.ds(i * 16, 16)] = zeros
        return 0

    lax.fori_loop(0, n // 16, body, 0)


def _deg_body(cols_hbm, dp0_hbm, dp1_hbm, colv, onesv, zerov, acc_sh):
    c = lax.axis_index("c")
    s = lax.axis_index("s")
    w = s * NC + c

    _zero_vmem(zerov, ROWS_PER_TILE)
    pltpu.sync_copy(zerov.at[pl.ds(0, ROWS_PER_TILE)],
                    acc_sh.at[pl.ds(s * ROWS_PER_TILE, ROWS_PER_TILE)])

    def fill(i, _):
        onesv[pl.ds(i * 16, 16)] = jnp.ones((16,), jnp.float32)
        return 0

    lax.fori_loop(0, EB // 16, fill, 0)
    pltpu.sync_copy(cols_hbm.at[pl.ds(w * DEG_BATCHES, DEG_BATCHES)], colv)
    plsc.subcore_barrier()

    def body(b, _):
        pltpu.sync_copy(onesv, acc_sh.at[colv.at[b]], add=True)
        return 0

    lax.fori_loop(0, DEG_BATCHES, body, 0)
    plsc.subcore_barrier()

    sl = pl.ds(s * ROWS_PER_TILE, ROWS_PER_TILE)

    @pl.when(c == 0)
    def _():
        pltpu.sync_copy(acc_sh.at[sl], dp0_hbm.at[sl])

    @pl.when(c == 1)
    def _():
        pltpu.sync_copy(acc_sh.at[sl], dp1_hbm.at[sl])


@functools.lru_cache(maxsize=None)
def _deg_kernel():
    return pl.kernel(
        _deg_body,
        out_type=[jax.ShapeDtypeStruct((NPAD,), jnp.float32),
                  jax.ShapeDtypeStruct((NPAD,), jnp.float32)],
        mesh=_mesh(),
        scratch_types=[pltpu.VMEM((DEG_BATCHES, EB), jnp.int32),
                       pltpu.VMEM((EB,), jnp.float32),
                       pltpu.VMEM((ROWS_PER_TILE,), jnp.float32),
                       pltpu.VMEM_SHARED((NPAD,), jnp.float32)],
    )


CH = 16


def _agg_body(rows2_hbm, cols_hbm, ycat_hbm, aggcat_hbm,
              rowcA, rowcB, colsv, gbuf0, gbuf1, acc_sh, gs0, gs1, ss0, ss1):
    c = lax.axis_index("c")
    s = lax.axis_index("s")

    zeros = jnp.zeros((16,), jnp.float32)

    def zrow(i, _):
        for k in range(C // 16):
            gbuf0[i, pl.ds(k * 16, 16)] = zeros
        return 0

    lax.fori_loop(0, EB, zrow, 0)

    def zb(k, _):
        pltpu.sync_copy(gbuf0,
                        acc_sh.at[pl.ds(s * ROWS_PER_TILE + k * EB, EB)])
        return 0

    lax.fori_loop(0, ROWS_PER_TILE // EB, zb, 0)

    pltpu.sync_copy(cols_hbm.at[pl.ds(s * AGG_BATCHES, AGG_BATCHES)], colsv)
    rbase = c * EIDX_ROWS + s * AGG_BATCHES
    rowbufs = [rowcA, rowcB]
    pltpu.sync_copy(rows2_hbm.at[pl.ds(rbase, CH)], rowcA)
    plsc.subcore_barrier()

    gb = [gbuf0, gbuf1]
    gsem = [gs0, gs1]
    ssem = [ss0, ss1]

    def gather(b):
        q, r = divmod(b, CH)
        return pltpu.async_copy(ycat_hbm.at[rowbufs[q % 2].at[r]],
                                gb[b % 2], gsem[b % 2])

    def scatter(b):
        return pltpu.async_copy(gb[b % 2], acc_sh.at[colsv.at[b]],
                                ssem[b % 2], add=True)

    g = {0: gather(0), 1: gather(1)}
    sd = {}
    for b in range(AGG_BATCHES):
        nb = b + 2
        if nb < AGG_BATCHES and nb % CH == 0:
            q = nb // CH
            pltpu.sync_copy(rows2_hbm.at[pl.ds(rbase + q * CH, CH)],
                            rowbufs[q % 2])
        g[b].wait()
        sd[b] = scatter(b)
        if nb < AGG_BATCHES:
            sd[b].wait()
            g[nb] = gather(nb)
    sd[AGG_BATCHES - 2].wait()
    sd[AGG_BATCHES - 1].wait()
    plsc.subcore_barrier()

    pltpu.sync_copy(acc_sh.at[pl.ds(s * ROWS_PER_TILE, ROWS_PER_TILE)],
                    aggcat_hbm.at[pl.ds(c * NPAD + s * ROWS_PER_TILE,
                                        ROWS_PER_TILE)])


@functools.lru_cache(maxsize=None)
def _agg_kernel():
    return pl.kernel(
        _agg_body,
        out_type=[jax.ShapeDtypeStruct((2 * NPAD, C), jnp.float32)],
        mesh=_mesh(),
        scratch_types=[pltpu.VMEM((CH, EB), jnp.int32),
                       pltpu.VMEM((CH, EB), jnp.int32),
                       pltpu.VMEM((AGG_BATCHES, EB), jnp.int32),
                       pltpu.VMEM((EB, C), jnp.float32),
                       pltpu.VMEM((EB, C), jnp.float32),
                       pltpu.VMEM_SHARED((NPAD, C), jnp.float32),
                       pltpu.SemaphoreType.DMA,
                       pltpu.SemaphoreType.DMA,
                       pltpu.SemaphoreType.DMA,
                       pltpu.SemaphoreType.DMA],
    )


EB2 = 64
DEC_BATCHES = EPAD // NW // EB2
DEC_ROWS = EPAD // EB2
DEC_CH = 40


def _dec_body(z_hbm, src_hbm, dst_hbm, r_hbm,
              srcv, dstv, sb0, db0, sb1, db1, pacc, rbuf, z_sh,
              sem_s0, sem_d0, sem_s1, sem_d1):
    c = lax.axis_index("c")
    s = lax.axis_index("s")
    w = s * NC + c

    sl = pl.ds(s * ROWS_PER_TILE, ROWS_PER_TILE)
    pltpu.sync_copy(z_hbm.at[sl], z_sh.at[sl])
    plsc.subcore_barrier()

    lanes = lax.iota(jnp.int32, 16)
    obase = w * DEC_BATCHES * EB2

    def compute(b, sbuf, dbuf):
        def edge(e, _):
            acc = (sbuf[e, pl.ds(0, 16)] * dbuf[e, pl.ds(0, 16)])
            for k in range(1, C // 16):
                acc = acc + sbuf[e, pl.ds(k * 16, 16)] * dbuf[e, pl.ds(k * 16, 16)]
            pacc[pl.ds(e * 16, 16)] = acc
            return 0

        lax.fori_loop(0, EB2, edge, 0, unroll=4)

        for g in range(EB2 // 16):
            parts = []
            for p in range(4):
                racc = plsc.load_gather(
                    pacc, [g * 256 + lanes * 16 + 4 * p])
                for l in range(1, 4):
                    racc = racc + plsc.load_gather(
                        pacc, [g * 256 + lanes * 16 + 4 * p + l])
                parts.append(racc)
            rbuf[pl.ds(g * 16, 16)] = (parts[0] + parts[1]) + (parts[2] + parts[3])
        pltpu.sync_copy(rbuf, r_hbm.at[pl.ds(obase + b * EB2, EB2)])

    def chunk(q, _):
        pltpu.sync_copy(
            src_hbm.at[pl.ds(w * DEC_BATCHES + q * DEC_CH, DEC_CH)], srcv)
        pltpu.sync_copy(
            dst_hbm.at[pl.ds(w * DEC_BATCHES + q * DEC_CH, DEC_CH)], dstv)

        def body(t2, _):
            lb = 2 * t2
            b = q * DEC_CH + lb
            g_s0 = pltpu.async_copy(z_sh.at[srcv.at[lb]], sb0, sem_s0)
            g_d0 = pltpu.async_copy(z_sh.at[dstv.at[lb]], db0, sem_d0)
            g_s1 = pltpu.async_copy(z_sh.at[srcv.at[lb + 1]], sb1, sem_s1)
            g_d1 = pltpu.async_copy(z_sh.at[dstv.at[lb + 1]], db1, sem_d1)
            g_s0.wait()
            g_d0.wait()
            compute(b, sb0, db0)
            g_s1.wait()
            g_d1.wait()
            compute(b + 1, sb1, db1)
            return 0

        lax.fori_loop(0, DEC_CH // 2, body, 0)
        return 0

    lax.fori_loop(0, DEC_BATCHES // DEC_CH, chunk, 0)


@functools.lru_cache(maxsize=None)
def _dec_kernel():
    return pl.kernel(
        _dec_body,
        out_type=[jax.ShapeDtypeStruct((EPAD,), jnp.float32)],
        mesh=_mesh(),
        scratch_types=[pltpu.VMEM((DEC_CH, EB2), jnp.int32),
                       pltpu.VMEM((DEC_CH, EB2), jnp.int32),
                       pltpu.VMEM((EB2, C), jnp.float32),
                       pltpu.VMEM((EB2, C), jnp.float32),
                       pltpu.VMEM((EB2, C), jnp.float32),
                       pltpu.VMEM((EB2, C), jnp.float32),
                       pltpu.VMEM((EB2 * 16,), jnp.float32),
                       pltpu.VMEM((EB2,), jnp.float32),
                       pltpu.VMEM_SHARED((NPAD, C), jnp.float32),
                       pltpu.SemaphoreType.DMA,
                       pltpu.SemaphoreType.DMA,
                       pltpu.SemaphoreType.DMA,
                       pltpu.SemaphoreType.DMA],
        compiler_params=pltpu.CompilerParams(needs_layout_passes=False),
    )


RB = 1024
GRID = NPAD // RB
DR = RB // 128


def _tile_to_col(t):
    r0 = lax.broadcasted_iota(jnp.int32, (RB, DR), 0)
    q0 = lax.broadcasted_iota(jnp.int32, (RB, DR), 1)
    a = jnp.where(r0 // 128 == q0, 1.0, 0.0)
    c2d = jnp.dot(a, t, preferred_element_type=jnp.float32)
    r1 = lax.broadcasted_iota(jnp.int32, (RB, 128), 0)
    l1 = lax.broadcasted_iota(jnp.int32, (RB, 128), 1)
    sel = jnp.where(l1 == r1 % 128, c2d, 0.0)
    return jnp.sum(sel, axis=1, keepdims=True)


def _enc_body(x_ref, w_ref, dp0_ref, dp1_ref, y_ref, dinv_ref):
    deg = dp0_ref[...] + dp1_ref[...] + 1.0
    dinv = lax.rsqrt(deg)
    dinv_ref[...] = dinv
    xw = jnp.dot(x_ref[...], w_ref[...], preferred_element_type=jnp.float32)
    y_ref[...] = xw * _tile_to_col(dinv)


def _enc_call(x_pad, w_gcn, dp0r, dp1r):
    return pl.pallas_call(
        _enc_body,
        grid=(GRID, 2),
        in_specs=[
            pl.BlockSpec((RB, D), lambda i, h: (i, 0)),
            pl.BlockSpec((D, C), lambda i, h: (0, h)),
            pl.BlockSpec((DR, 128), lambda i, h: (i, 0)),
            pl.BlockSpec((DR, 128), lambda i, h: (i, 0)),
        ],
        out_specs=[
            pl.BlockSpec((RB, C), lambda i, h: (h * GRID + i, 0)),
            pl.BlockSpec((DR, 128), lambda i, h: (i, 0)),
        ],
        out_shape=[
            jax.ShapeDtypeStruct((2 * NPAD, C), jnp.float32),
            jax.ShapeDtypeStruct((NPAD // 128, 128), jnp.float32),
        ],
    )(x_pad, w_gcn, dp0r, dp1r)


def _gru_body(agg0_ref, agg1_ref, y0_ref, y1_ref, dinv_ref, hid_ref,
              wz0_ref, wz1_ref, wr0_ref, wr1_ref, wh0_ref, wh1_ref,
              wlin_ref, bias_ref, hn_ref, z_ref):
    dinv = _tile_to_col(dinv_ref[...])
    aggc = jnp.concatenate([agg0_ref[...], agg1_ref[...]], axis=1)
    yc = jnp.concatenate([y0_ref[...], y1_ref[...]], axis=1)
    b_gcn = bias_ref[0:1, :]
    h = dinv * (aggc + yc) + b_gcn

    hid = hid_ref[...]
    cat1 = jnp.concatenate([h, hid], axis=1)
    wz = wz0_ref[...] + wz1_ref[...]
    wr = wr0_ref[...] + wr1_ref[...]
    wh = wh0_ref[...] + wh1_ref[...]
    zg = jax.nn.sigmoid(
        jnp.dot(cat1, wz, preferred_element_type=jnp.float32) + bias_ref[1:2, :])
    rg = jax.nn.sigmoid(
        jnp.dot(cat1, wr, preferred_element_type=jnp.float32) + bias_ref[2:3, :])
    cat2 = jnp.concatenate([h, rg * hid], axis=1)
    ht = jnp.tanh(
        jnp.dot(cat2, wh, preferred_element_type=jnp.float32) + bias_ref[3:4, :])
    hn = zg * hid + (1.0 - zg) * ht
    hn_ref[...] = hn

    hr = jnp.maximum(hn, 0.0)
    logits = (jnp.dot(hr, wlin_ref[...], preferred_element_type=jnp.float32)
              + bias_ref[4:5, :C])
    m = jnp.max(logits, axis=1, keepdims=True)
    ez = jnp.exp(logits - m)
    z_ref[...] = ez / jnp.sum(ez, axis=1, keepdims=True)


def _gru_call(aggcat, ycat, dinvr, hid_pad,
              wz0, wz1, wr0, wr1, wh0, wh1, wlin, bias):
    full = lambda a, b: pl.BlockSpec((a, b), lambda i: (0, 0))
    return pl.pallas_call(
        _gru_body,
        grid=(GRID,),
        in_specs=[
            pl.BlockSpec((RB, C), lambda i: (i, 0)),
            pl.BlockSpec((RB, C), lambda i: (GRID + i, 0)),
            pl.BlockSpec((RB, C), lambda i: (i, 0)),
            pl.BlockSpec((RB, C), lambda i: (GRID + i, 0)),
            pl.BlockSpec((DR, 128), lambda i: (i, 0)),
            pl.BlockSpec((RB, S), lambda i: (i, 0)),
            full(D + S, S), full(D + S, S), full(D + S, S),
            full(D + S, S), full(D + S, S), full(D + S, S),
            full(S, C), full(8, 256),
        ],
        out_specs=[
            pl.BlockSpec((RB, S), lambda i: (i, 0)),
            pl.BlockSpec((RB, C), lambda i: (i, 0)),
        ],
        out_shape=[
            jax.ShapeDtypeStruct((NPAD, S), jnp.float32),
            jax.ShapeDtypeStruct((NPAD, C), jnp.float32),
        ],
    )(aggcat, aggcat, ycat, ycat, dinvr, hid_pad,
      wz0, wz1, wr0, wr1, wh0, wh1, wlin, bias)


def kernel(x, edge_index, edge_label_index, hidden, W_gcn, b_gcn,
           Wz0, Wz1, bz, Wr0, Wr1, br, Wh0, Wh1, bh, W_lin, b_lin):
    pad_e = jnp.arange(EPAD - E, dtype=jnp.int32)
    cols_p = jnp.concatenate(
        [edge_index[1], N + (pad_e % (NPAD - N))]).reshape(EIDX_ROWS, EB)
    rows_p = jnp.concatenate(
        [edge_index[0], pad_e % N]).reshape(EIDX_ROWS, EB)
    src_p = jnp.concatenate(
        [edge_label_index[0], pad_e % N]).reshape(DEC_ROWS, EB2)
    dst_p = jnp.concatenate(
        [edge_label_index[1], pad_e % N]).reshape(DEC_ROWS, EB2)

    x_pad = jnp.pad(x, ((0, NPAD - N), (0, 0)))
    hid_pad = jnp.pad(hidden, ((0, NPAD - N), (0, 0)))

    bias = jnp.zeros((8, 256), jnp.float32)
    bias = bias.at[0, :].set(b_gcn).at[1, :].set(bz).at[2, :].set(br)
    bias = bias.at[3, :].set(bh).at[4, :C].set(b_lin)

    rows2 = jnp.concatenate([rows_p, rows_p + NPAD])

    dp0, dp1 = _deg_kernel()(cols_p)
    dp0r = dp0.reshape(NPAD // 128, 128)
    dp1r = dp1.reshape(NPAD // 128, 128)

    ycat, dinvr = _enc_call(x_pad, W_gcn, dp0r, dp1r)
    (aggcat,) = _agg_kernel()(rows2, cols_p, ycat)
    hn, z = _gru_call(aggcat, ycat, dinvr, hid_pad,
                      Wz0, Wz1, Wr0, Wr1, Wh0, Wh1, W_lin, bias)
    (r_pad,) = _dec_kernel()(z, src_p, dst_p)

    return (r_pad[:EL], hn[:N])


# speedup vs baseline: 16.3554x; 1.0876x over previous
import functools

import jax
import jax.numpy as jnp
from jax import lax
from jax.experimental import pallas as pl
from jax.experimental.pallas import tpu as pltpu
from jax.experimental.pallas import tpu_sc as plsc

N = 10000
D = 256
S = 256
C = 128
E = 160000
EL = 160000

NC = 2
NS = 16
NW = NC * NS

NPAD = 10240
EPAD = 163840
ROWS_PER_TILE = NPAD // NS
EB = 128
DEG_BATCHES = EPAD // NW // EB
AGG_BATCHES = EPAD // NS // EB
DEC_CHUNK = 8
EIDX_ROWS = EPAD // EB


def _mesh():
    return plsc.VectorSubcoreMesh(core_axis_name="c", subcore_axis_name="s",
                                  num_cores=NC, num_subcores=NS)


def _zero_vmem(ref, n):
    zeros = jnp.zeros((16,), jnp.float32)

    def body(i, _):
        ref[pl.ds(i * 16, 16)] = zeros
        return 0

    lax.fori_loop(0, n // 16, body, 0)


def _deg_body(cols_hbm, dp0_hbm, dp1_hbm, colv, onesv, zerov, acc_sh):
    c = lax.axis_index("c")
    s = lax.axis_index("s")
    w = s * NC + c

    _zero_vmem(zerov, ROWS_PER_TILE)
    pltpu.sync_copy(zerov.at[pl.ds(0, ROWS_PER_TILE)],
                    acc_sh.at[pl.ds(s * ROWS_PER_TILE, ROWS_PER_TILE)])

    def fill(i, _):
        onesv[pl.ds(i * 16, 16)] = jnp.ones((16,), jnp.float32)
        return 0

    lax.fori_loop(0, EB // 16, fill, 0)
    pltpu.sync_copy(cols_hbm.at[pl.ds(w * DEG_BATCHES, DEG_BATCHES)], colv)
    plsc.subcore_barrier()

    def body(b, _):
        pltpu.sync_copy(onesv, acc_sh.at[colv.at[b]], add=True)
        return 0

    lax.fori_loop(0, DEG_BATCHES, body, 0)
    plsc.subcore_barrier()

    sl = pl.ds(s * ROWS_PER_TILE, ROWS_PER_TILE)

    @pl.when(c == 0)
    def _():
        pltpu.sync_copy(acc_sh.at[sl], dp0_hbm.at[sl])

    @pl.when(c == 1)
    def _():
        pltpu.sync_copy(acc_sh.at[sl], dp1_hbm.at[sl])


@functools.lru_cache(maxsize=None)
def _deg_kernel():
    return pl.kernel(
        _deg_body,
        out_type=[jax.ShapeDtypeStruct((NPAD,), jnp.float32),
                  jax.ShapeDtypeStruct((NPAD,), jnp.float32)],
        mesh=_mesh(),
        scratch_types=[pltpu.VMEM((DEG_BATCHES, EB), jnp.int32),
                       pltpu.VMEM((EB,), jnp.float32),
                       pltpu.VMEM((ROWS_PER_TILE,), jnp.float32),
                       pltpu.VMEM_SHARED((NPAD,), jnp.float32)],
    )


CH = 16


def _agg_body(rows2_hbm, cols_hbm, ycat_hbm, aggcat_hbm,
              rowcA, rowcB, colsv, gbuf0, gbuf1, acc_sh, gs0, gs1, ss0, ss1):
    c = lax.axis_index("c")
    s = lax.axis_index("s")

    zeros = jnp.zeros((16,), jnp.float32)

    def zrow(i, _):
        for k in range(C // 16):
            gbuf0[i, pl.ds(k * 16, 16)] = zeros
        return 0

    lax.fori_loop(0, EB, zrow, 0)

    def zb(k, _):
        pltpu.sync_copy(gbuf0,
                        acc_sh.at[pl.ds(s * ROWS_PER_TILE + k * EB, EB)])
        return 0

    lax.fori_loop(0, ROWS_PER_TILE // EB, zb, 0)

    pltpu.sync_copy(cols_hbm.at[pl.ds(s * AGG_BATCHES, AGG_BATCHES)], colsv)
    rbase = c * EIDX_ROWS + s * AGG_BATCHES
    rowbufs = [rowcA, rowcB]
    pltpu.sync_copy(rows2_hbm.at[pl.ds(rbase, CH)], rowcA)
    plsc.subcore_barrier()

    gb = [gbuf0, gbuf1]
    gsem = [gs0, gs1]
    ssem = [ss0, ss1]

    def gather(b):
        q, r = divmod(b, CH)
        return pltpu.async_copy(ycat_hbm.at[rowbufs[q % 2].at[r]],
                                gb[b % 2], gsem[b % 2])

    def scatter(b):
        return pltpu.async_copy(gb[b % 2], acc_sh.at[colsv.at[b]],
                                ssem[b % 2], add=True)

    g = {0: gather(0), 1: gather(1)}
    sd = {}
    for b in range(AGG_BATCHES):
        nb = b + 2
        if nb < AGG_BATCHES and nb % CH == 0:
            q = nb // CH
            pltpu.sync_copy(rows2_hbm.at[pl.ds(rbase + q * CH, CH)],
                            rowbufs[q % 2])
        g[b].wait()
        sd[b] = scatter(b)
        if nb < AGG_BATCHES:
            sd[b].wait()
            g[nb] = gather(nb)
    sd[AGG_BATCHES - 2].wait()
    sd[AGG_BATCHES - 1].wait()
    plsc.subcore_barrier()

    pltpu.sync_copy(acc_sh.at[pl.ds(s * ROWS_PER_TILE, ROWS_PER_TILE)],
                    aggcat_hbm.at[pl.ds(c * NPAD + s * ROWS_PER_TILE,
                                        ROWS_PER_TILE)])


@functools.lru_cache(maxsize=None)
def _agg_kernel():
    return pl.kernel(
        _agg_body,
        out_type=[jax.ShapeDtypeStruct((2 * NPAD, C), jnp.float32)],
        mesh=_mesh(),
        scratch_types=[pltpu.VMEM((CH, EB), jnp.int32),
                       pltpu.VMEM((CH, EB), jnp.int32),
                       pltpu.VMEM((AGG_BATCHES, EB), jnp.int32),
                       pltpu.VMEM((EB, C), jnp.float32),
                       pltpu.VMEM((EB, C), jnp.float32),
                       pltpu.VMEM_SHARED((NPAD, C), jnp.float32),
                       pltpu.SemaphoreType.DMA,
                       pltpu.SemaphoreType.DMA,
                       pltpu.SemaphoreType.DMA,
                       pltpu.SemaphoreType.DMA],
    )


EB2 = 64
DEC_BATCHES = EPAD // NW // EB2
DEC_ROWS = EPAD // EB2
DEC_CH = 40


def _dec_body(z_hbm, src_hbm, dst_hbm, r_hbm,
              srcv, dstv, sb0, db0, sb1, db1, pacc, rbuf, z_sh,
              sem_s0, sem_d0, sem_s1, sem_d1):
    c = lax.axis_index("c")
    s = lax.axis_index("s")
    w = s * NC + c

    sl = pl.ds(s * ROWS_PER_TILE, ROWS_PER_TILE)
    pltpu.sync_copy(z_hbm.at[sl], z_sh.at[sl])
    plsc.subcore_barrier()

    lanes = lax.iota(jnp.int32, 16)
    obase = w * DEC_BATCHES * EB2

    def compute(b, sbuf, dbuf):
        def edge(e, _):
            acc = (sbuf[e, pl.ds(0, 16)] * dbuf[e, pl.ds(0, 16)])
            for k in range(1, C // 16):
                acc = acc + sbuf[e, pl.ds(k * 16, 16)] * dbuf[e, pl.ds(k * 16, 16)]
            pacc[pl.ds(e * 16, 16)] = acc
            return 0

        lax.fori_loop(0, EB2, edge, 0, unroll=4)

        for g in range(EB2 // 16):
            parts = []
            for p in range(4):
                racc = plsc.load_gather(
                    pacc, [g * 256 + lanes * 16 + 4 * p])
                for l in range(1, 4):
                    racc = racc + plsc.load_gather(
                        pacc, [g * 256 + lanes * 16 + 4 * p + l])
                parts.append(racc)
            rbuf[pl.ds(g * 16, 16)] = (parts[0] + parts[1]) + (parts[2] + parts[3])
        pltpu.sync_copy(rbuf, r_hbm.at[pl.ds(obase + b * EB2, EB2)])

    def issue(bv, sbuf, dbuf, ssem, dsem):
        pltpu.async_copy(z_sh.at[srcv.at[bv]], sbuf, ssem)
        pltpu.async_copy(z_sh.at[dstv.at[bv]], dbuf, dsem)

    def drain(sbuf, dbuf, ssem, dsem):
        pltpu.make_async_copy(z_hbm.at[pl.ds(0, EB2)], sbuf, ssem).wait()
        pltpu.make_async_copy(z_hbm.at[pl.ds(0, EB2)], dbuf, dsem).wait()

    def chunk(q, _):
        pltpu.sync_copy(
            src_hbm.at[pl.ds(w * DEC_BATCHES + q * DEC_CH, DEC_CH)], srcv)
        pltpu.sync_copy(
            dst_hbm.at[pl.ds(w * DEC_BATCHES + q * DEC_CH, DEC_CH)], dstv)

        issue(0, sb0, db0, sem_s0, sem_d0)
        issue(1, sb1, db1, sem_s1, sem_d1)

        def body(t2, _):
            lb = 2 * t2
            b = q * DEC_CH + lb
            drain(sb0, db0, sem_s0, sem_d0)
            compute(b, sb0, db0)
            issue(jnp.minimum(lb + 2, DEC_CH - 1), sb0, db0, sem_s0, sem_d0)
            drain(sb1, db1, sem_s1, sem_d1)
            compute(b + 1, sb1, db1)
            issue(jnp.minimum(lb + 3, DEC_CH - 1), sb1, db1, sem_s1, sem_d1)
            return 0

        lax.fori_loop(0, DEC_CH // 2, body, 0)
        drain(sb0, db0, sem_s0, sem_d0)
        drain(sb1, db1, sem_s1, sem_d1)
        return 0

    lax.fori_loop(0, DEC_BATCHES // DEC_CH, chunk, 0)


@functools.lru_cache(maxsize=None)
def _dec_kernel():
    return pl.kernel(
        _dec_body,
        out_type=[jax.ShapeDtypeStruct((EPAD,), jnp.float32)],
        mesh=_mesh(),
        scratch_types=[pltpu.VMEM((DEC_CH, EB2), jnp.int32),
                       pltpu.VMEM((DEC_CH, EB2), jnp.int32),
                       pltpu.VMEM((EB2, C), jnp.float32),
                       pltpu.VMEM((EB2, C), jnp.float32),
                       pltpu.VMEM((EB2, C), jnp.float32),
                       pltpu.VMEM((EB2, C), jnp.float32),
                       pltpu.VMEM((EB2 * 16,), jnp.float32),
                       pltpu.VMEM((EB2,), jnp.float32),
                       pltpu.VMEM_SHARED((NPAD, C), jnp.float32),
                       pltpu.SemaphoreType.DMA,
                       pltpu.SemaphoreType.DMA,
                       pltpu.SemaphoreType.DMA,
                       pltpu.SemaphoreType.DMA],
        compiler_params=pltpu.CompilerParams(needs_layout_passes=False),
    )


RB = 1024
GRID = NPAD // RB
DR = RB // 128


def _tile_to_col(t):
    r0 = lax.broadcasted_iota(jnp.int32, (RB, DR), 0)
    q0 = lax.broadcasted_iota(jnp.int32, (RB, DR), 1)
    a = jnp.where(r0 // 128 == q0, 1.0, 0.0)
    c2d = jnp.dot(a, t, preferred_element_type=jnp.float32)
    r1 = lax.broadcasted_iota(jnp.int32, (RB, 128), 0)
    l1 = lax.broadcasted_iota(jnp.int32, (RB, 128), 1)
    sel = jnp.where(l1 == r1 % 128, c2d, 0.0)
    return jnp.sum(sel, axis=1, keepdims=True)


def _enc_body(x_ref, w_ref, dp0_ref, dp1_ref, y_ref, dinv_ref):
    deg = dp0_ref[...] + dp1_ref[...] + 1.0
    dinv = lax.rsqrt(deg)
    dinv_ref[...] = dinv
    xw = jnp.dot(x_ref[...], w_ref[...], preferred_element_type=jnp.float32)
    y_ref[...] = xw * _tile_to_col(dinv)


def _enc_call(x_pad, w_gcn, dp0r, dp1r):
    return pl.pallas_call(
        _enc_body,
        grid=(GRID, 2),
        in_specs=[
            pl.BlockSpec((RB, D), lambda i, h: (i, 0)),
            pl.BlockSpec((D, C), lambda i, h: (0, h)),
            pl.BlockSpec((DR, 128), lambda i, h: (i, 0)),
            pl.BlockSpec((DR, 128), lambda i, h: (i, 0)),
        ],
        out_specs=[
            pl.BlockSpec((RB, C), lambda i, h: (h * GRID + i, 0)),
            pl.BlockSpec((DR, 128), lambda i, h: (i, 0)),
        ],
        out_shape=[
            jax.ShapeDtypeStruct((2 * NPAD, C), jnp.float32),
            jax.ShapeDtypeStruct((NPAD // 128, 128), jnp.float32),
        ],
    )(x_pad, w_gcn, dp0r, dp1r)


def _gru_body(agg0_ref, agg1_ref, y0_ref, y1_ref, dinv_ref, hid_ref,
              wz0_ref, wz1_ref, wr0_ref, wr1_ref, wh0_ref, wh1_ref,
              wlin_ref, bias_ref, hn_ref, z_ref):
    dinv = _tile_to_col(dinv_ref[...])
    aggc = jnp.concatenate([agg0_ref[...], agg1_ref[...]], axis=1)
    yc = jnp.concatenate([y0_ref[...], y1_ref[...]], axis=1)
    b_gcn = bias_ref[0:1, :]
    h = dinv * (aggc + yc) + b_gcn

    hid = hid_ref[...]
    cat1 = jnp.concatenate([h, hid], axis=1)
    wz = wz0_ref[...] + wz1_ref[...]
    wr = wr0_ref[...] + wr1_ref[...]
    wh = wh0_ref[...] + wh1_ref[...]
    zg = jax.nn.sigmoid(
        jnp.dot(cat1, wz, preferred_element_type=jnp.float32) + bias_ref[1:2, :])
    rg = jax.nn.sigmoid(
        jnp.dot(cat1, wr, preferred_element_type=jnp.float32) + bias_ref[2:3, :])
    cat2 = jnp.concatenate([h, rg * hid], axis=1)
    ht = jnp.tanh(
        jnp.dot(cat2, wh, preferred_element_type=jnp.float32) + bias_ref[3:4, :])
    hn = zg * hid + (1.0 - zg) * ht
    hn_ref[...] = hn

    hr = jnp.maximum(hn, 0.0)
    logits = (jnp.dot(hr, wlin_ref[...], preferred_element_type=jnp.float32)
              + bias_ref[4:5, :C])
    m = jnp.max(logits, axis=1, keepdims=True)
    ez = jnp.exp(logits - m)
    z_ref[...] = ez / jnp.sum(ez, axis=1, keepdims=True)


def _gru_call(aggcat, ycat, dinvr, hid_pad,
              wz0, wz1, wr0, wr1, wh0, wh1, wlin, bias):
    full = lambda a, b: pl.BlockSpec((a, b), lambda i: (0, 0))
    return pl.pallas_call(
        _gru_body,
        grid=(GRID,),
        in_specs=[
            pl.BlockSpec((RB, C), lambda i: (i, 0)),
            pl.BlockSpec((RB, C), lambda i: (GRID + i, 0)),
            pl.BlockSpec((RB, C), lambda i: (i, 0)),
            pl.BlockSpec((RB, C), lambda i: (GRID + i, 0)),
            pl.BlockSpec((DR, 128), lambda i: (i, 0)),
            pl.BlockSpec((RB, S), lambda i: (i, 0)),
            full(D + S, S), full(D + S, S), full(D + S, S),
            full(D + S, S), full(D + S, S), full(D + S, S),
            full(S, C), full(8, 256),
        ],
        out_specs=[
            pl.BlockSpec((RB, S), lambda i: (i, 0)),
            pl.BlockSpec((RB, C), lambda i: (i, 0)),
        ],
        out_shape=[
            jax.ShapeDtypeStruct((NPAD, S), jnp.float32),
            jax.ShapeDtypeStruct((NPAD, C), jnp.float32),
        ],
    )(aggcat, aggcat, ycat, ycat, dinvr, hid_pad,
      wz0, wz1, wr0, wr1, wh0, wh1, wlin, bias)


def kernel(x, edge_index, edge_label_index, hidden, W_gcn, b_gcn,
           Wz0, Wz1, bz, Wr0, Wr1, br, Wh0, Wh1, bh, W_lin, b_lin):
    pad_e = jnp.arange(EPAD - E, dtype=jnp.int32)
    cols_p = jnp.concatenate(
        [edge_index[1], N + (pad_e % (NPAD - N))]).reshape(EIDX_ROWS, EB)
    rows_p = jnp.concatenate(
        [edge_index[0], pad_e % N]).reshape(EIDX_ROWS, EB)
    src_p = jnp.concatenate(
        [edge_label_index[0], pad_e % N]).reshape(DEC_ROWS, EB2)
    dst_p = jnp.concatenate(
        [edge_label_index[1], pad_e % N]).reshape(DEC_ROWS, EB2)

    x_pad = jnp.pad(x, ((0, NPAD - N), (0, 0)))
    hid_pad = jnp.pad(hidden, ((0, NPAD - N), (0, 0)))

    bias = jnp.zeros((8, 256), jnp.float32)
    bias = bias.at[0, :].set(b_gcn).at[1, :].set(bz).at[2, :].set(br)
    bias = bias.at[3, :].set(bh).at[4, :C].set(b_lin)

    rows2 = jnp.concatenate([rows_p, rows_p + NPAD])

    dp0, dp1 = _deg_kernel()(cols_p)
    dp0r = dp0.reshape(NPAD // 128, 128)
    dp1r = dp1.reshape(NPAD // 128, 128)

    ycat, dinvr = _enc_call(x_pad, W_gcn, dp0r, dp1r)
    (aggcat,) = _agg_kernel()(rows2, cols_p, ycat)
    hn, z = _gru_call(aggcat, ycat, dinvr, hid_pad,
                      Wz0, Wz1, Wr0, Wr1, Wh0, Wh1, W_lin, bias)
    (r_pad,) = _dec_kernel()(z, src_p, dst_p)

    return (r_pad[:EL], hn[:N])
